# baseline (speedup 1.0000x reference)
import functools

import jax
import jax.numpy as jnp
from jax import lax
from jax.experimental import pallas as pl
from jax.experimental.pallas import tpu as pltpu
from jax.experimental.pallas import tpu_sc as plsc

N = 10000
E = 320000
D = 128
R = 5
B = 512
H = 256

NPK = 10240
PK = NPK // 4
T5 = R * NPK
EROWS = E // 128
EROWS_PAD = 2560
ROWS_PER_W = 80
TILE_ROWS = T5 // 16
WB_ROWS = NPK // 16
PB = 256
NBLK = PK // PB


@functools.cache
def _sc_mesh():
    return plsc.VectorSubcoreMesh(core_axis_name="c", subcore_axis_name="s")


def _idx_body(src_ref, dst_ref, et_ref, i0_ref, isrc_ref, idst_ref):
    et = et_ref[...]
    i0_ref[...] = et * NPK + src_ref[...]
    isrc_ref[...] = src_ref[...]
    idst_ref[...] = et * NPK + dst_ref[...]


def _edge_indices(src_p, dst_p, et_p):
    return pl.pallas_call(
        _idx_body,
        out_shape=(
            jax.ShapeDtypeStruct((EROWS_PAD, 128), jnp.int32),
            jax.ShapeDtypeStruct((EROWS_PAD, 128), jnp.int32),
            jax.ShapeDtypeStruct((EROWS_PAD, 128), jnp.int32),
        ),
    )(src_p, dst_p, et_p)


def _basis_w(comp_ref, bases_ref, r):
    w = comp_ref[r, 0] * bases_ref[0]
    for b in range(1, 4):
        w = w + comp_ref[r, b] * bases_ref[b]
    return w


def _layer0_body(x_ref, comp_ref, bases_ref, root_ref, bias_ref,
                 hall_ref, self_ref):
    x = x_ref[...]
    for r in range(R):
        w = _basis_w(comp_ref, bases_ref, r)
        hall_ref[r] = jnp.dot(x, w, preferred_element_type=jnp.float32)
    self_ref[...] = (
        jnp.dot(x, root_ref[...], preferred_element_type=jnp.float32)
        + bias_ref[...])


def _layer0(x, comp, bases, root, bias_row, rb=2048):
    return pl.pallas_call(
        _layer0_body,
        grid=(NPK // rb,),
        in_specs=[
            pl.BlockSpec((rb, D), lambda i: (i, 0)),
            pl.BlockSpec(memory_space=pltpu.MemorySpace.SMEM),
            pl.BlockSpec((4, D, 32), lambda i: (0, 0, 0)),
            pl.BlockSpec((D, 32), lambda i: (0, 0)),
            pl.BlockSpec((1, 32), lambda i: (0, 0)),
        ],
        out_specs=(
            pl.BlockSpec((R, rb, 32), lambda i: (0, i, 0)),
            pl.BlockSpec((rb, 32), lambda i: (i, 0)),
        ),
        out_shape=(
            jax.ShapeDtypeStruct((R, NPK, 32), jnp.float32),
            jax.ShapeDtypeStruct((NPK, 32), jnp.float32),
        ),
    )(x, comp, bases, root, bias_row)


def _sc_count(idst, ones32, zeros32):
    @functools.partial(
        pl.kernel,
        out_type=jax.ShapeDtypeStruct((2, R, NPK, 32), jnp.float32),
        mesh=_sc_mesh(),
        scratch_types=[
            pltpu.VMEM((ROWS_PER_W, 128), jnp.int32),
            pltpu.VMEM((128, 32), jnp.float32),
            pltpu.VMEM_SHARED((T5, 32), jnp.float32),
            pltpu.SemaphoreType.DMA,
            pltpu.SemaphoreType.DMA,
        ],
        compiler_params=pltpu.CompilerParams(use_tc_tiling_on_sc=False),
    )
    def k(idst_hbm, ones_hbm, zeros_hbm, out_hbm, idst_v, ones_v, cnt_sp,
          sem, ssem):
        cid = lax.axis_index("c")
        sid = lax.axis_index("s")
        wid = sid * 2 + cid
        pltpu.async_copy(zeros_hbm, cnt_sp.at[pl.ds(sid * TILE_ROWS, TILE_ROWS)],
                         sem).wait()
        pltpu.async_copy(idst_hbm.at[pl.ds(wid * ROWS_PER_W, ROWS_PER_W)],
                         idst_v, sem).wait()
        pltpu.async_copy(ones_hbm, ones_v, sem).wait()
        plsc.subcore_barrier()

        def body(j, carry):
            for u in range(5):
                pltpu.async_copy(ones_v, cnt_sp.at[idst_v.at[j * 5 + u]], ssem,
                                 add=True)
            return carry

        lax.fori_loop(0, ROWS_PER_W // 5, body, 0)

        def drain(j, carry):
            pltpu.make_async_copy(ones_hbm, ones_v, ssem).wait()
            return carry

        lax.fori_loop(0, ROWS_PER_W, drain, 0)
        plsc.subcore_barrier()
        for r in range(R):
            pltpu.async_copy(
                cnt_sp.at[pl.ds(r * NPK + sid * WB_ROWS, WB_ROWS)],
                out_hbm.at[cid, r].at[pl.ds(sid * WB_ROWS, WB_ROWS)],
                sem).wait()

    return k(idst, ones32, zeros32)


def _sc_edge(tab, idxg, idst, zeros32):
    @functools.partial(
        pl.kernel,
        out_type=jax.ShapeDtypeStruct((1, R, NPK, 32), jnp.float32),
        mesh=_sc_mesh(),
        scratch_types=[
            pltpu.VMEM((20, 128), jnp.int32),
            pltpu.VMEM((20, 128), jnp.int32),
            [pltpu.VMEM((128, 32), jnp.float32) for _ in range(5)],
            [pltpu.SemaphoreType.DMA for _ in range(5)],
            [pltpu.SemaphoreType.DMA for _ in range(5)],
            pltpu.VMEM_SHARED((T5, 32), jnp.float32),
            pltpu.SemaphoreType.DMA,
        ],
        compiler_params=pltpu.CompilerParams(use_tc_tiling_on_sc=False),
    )
    def k(tab_hbm, idxg_hbm, idst_hbm, zeros_hbm, out_hbm,
          idxg_v, idst_v, msgs, gsems, ssems, k_acc, sem):
        cid = lax.axis_index("c")
        sid = lax.axis_index("s")
        base = sid * 160
        trips = 32

        def stage(k4):
            a = pltpu.async_copy(
                idxg_hbm.at[pl.ds(base + k4 * 20, 20)], idxg_v, sem)
            b = pltpu.async_copy(
                idst_hbm.at[pl.ds(base + k4 * 20, 20)], idst_v, sem)
            a.wait()
            b.wait()

        def swait(sem):
            pltpu.make_async_copy(zeros_hbm.at[pl.ds(0, 128)], msgs[0],
                                  sem).wait()

        def fire_gathers(rk):
            for j in range(5):
                pltpu.async_copy(tab_hbm.at[idxg_v.at[rk * 5 + j]], msgs[j],
                                 gsems[j])

        def fire_scatters(rk):
            for j in range(5):
                swait(gsems[j])
                pltpu.async_copy(msgs[j], k_acc.at[idst_v.at[rk * 5 + j]],
                                 ssems[j], add=True)

        @pl.when(cid == 0)
        def _():
            zero_desc = pltpu.async_copy(
                zeros_hbm, k_acc.at[pl.ds(sid * TILE_ROWS, TILE_ROWS)], sem)
            stage(0)
            fire_gathers(0)
            zero_desc.wait()
            plsc.subcore_barrier()
            fire_scatters(0)

            def body(kk, carry):
                rk = lax.rem(kk, 4)

                @pl.when(rk == 0)
                def _():
                    for j in range(5):
                        swait(ssems[j])
                    stage(lax.div(kk, 4))

                for j in range(5):
                    @pl.when(rk != 0)
                    def _(j=j):
                        swait(ssems[j])
                fire_gathers(rk)
                fire_scatters(rk)
                return carry

            lax.fori_loop(1, trips, body, 0)
            for j in range(5):
                swait(ssems[j])
            plsc.subcore_barrier()
            wdescs = [
                pltpu.async_copy(
                    k_acc.at[pl.ds(r * NPK + sid * WB_ROWS, WB_ROWS)],
                    out_hbm.at[0, r].at[pl.ds(sid * WB_ROWS, WB_ROWS)],
                    sem)
                for r in range(R)
            ]
            for d in wdescs:
                d.wait()

    return k(tab, idxg, idst, zeros32)


def _blkdiag(w, zeros):
    rows = [jnp.concatenate([w if q == q2 else zeros for q in range(4)], axis=1)
            for q2 in range(4)]
    return jnp.concatenate(rows, axis=0)


def _post1_body(self_ref, acc_ref, cnt_ref, h_ref, scale_ref):
    cn = cnt_ref[...]
    scale = 1.0 / jnp.maximum(cn[0] + cn[1], 1.0)
    scale_ref[...] = scale
    acc_s = acc_ref[...][0]
    agg = (acc_s * scale).sum(axis=0)
    h_ref[...] = jnp.tanh(self_ref[...] + agg)


def _post1(self0p, accp, cntp):
    return pl.pallas_call(
        _post1_body,
        grid=(NBLK,),
        in_specs=[
            pl.BlockSpec((PB, 128), lambda i: (i, 0)),
            pl.BlockSpec((1, R, PB, 128), lambda i: (0, 0, i, 0)),
            pl.BlockSpec((2, R, PB, 128), lambda i: (0, 0, i, 0)),
        ],
        out_specs=(
            pl.BlockSpec((PB, 128), lambda i: (i, 0)),
            pl.BlockSpec((R, PB, 128), lambda i: (0, i, 0)),
        ),
        out_shape=(
            jax.ShapeDtypeStruct((PK, 128), jnp.float32),
            jax.ShapeDtypeStruct((R, PK, 128), jnp.float32),
        ),
    )(self0p, accp, cntp)


def _post23_body(hprev_ref, acc_ref, scale_ref, comp_ref, bases_ref,
                 root_ref, bias_ref, h_ref):
    z32 = jnp.zeros((32, 32), jnp.float32)
    acc_s = acc_ref[...][0]
    scale = scale_ref[...]
    contrib = None
    for r in range(R):
        w = _basis_w(comp_ref, bases_ref, r)
        wb = _blkdiag(w, z32)
        term = jnp.dot(acc_s[r] * scale[r], wb,
                       preferred_element_type=jnp.float32)
        contrib = term if contrib is None else contrib + term
    rootb = _blkdiag(root_ref[...], z32)
    bias = bias_ref[...]
    biaspk = jnp.concatenate([bias, bias, bias, bias], axis=1)
    h_ref[...] = jnp.tanh(
        jnp.dot(hprev_ref[...], rootb, preferred_element_type=jnp.float32)
        + biaspk + contrib)


def _post23(hprevp, accp, scalep, comp, bases, root, bias_row):
    return pl.pallas_call(
        _post23_body,
        grid=(NBLK,),
        in_specs=[
            pl.BlockSpec((PB, 128), lambda i: (i, 0)),
            pl.BlockSpec((1, R, PB, 128), lambda i: (0, 0, i, 0)),
            pl.BlockSpec((R, PB, 128), lambda i: (0, i, 0)),
            pl.BlockSpec(memory_space=pltpu.MemorySpace.SMEM),
            pl.BlockSpec((4, 32, 32), lambda i: (0, 0, 0)),
            pl.BlockSpec((32, 32), lambda i: (0, 0)),
            pl.BlockSpec((1, 32), lambda i: (0, 0)),
        ],
        out_specs=pl.BlockSpec((PB, 128), lambda i: (i, 0)),
        out_shape=jax.ShapeDtypeStruct((PK, 128), jnp.float32),
    )(hprevp, accp, scalep, comp, bases, root, bias_row)


def _head_body(h1_ref, h2_ref, h3_ref, acc_ref, scale_ref,
               comp_ref, bases_ref, root_ref, bias_ref,
               wih0_ref, whh0_ref, b0_ref,
               wih1_ref, whh1_ref, b1_ref,
               l1w_ref, l1b_ref, l2w_ref, l2b_ref,
               out_ref, gx3_ref, gx2_ref, ys_ref):
    z32 = jnp.zeros((32, 32), jnp.float32)
    acc_s = acc_ref[...][0]
    scale = scale_ref[...]
    contrib = None
    for r in range(R):
        w = _basis_w(comp_ref, bases_ref, r)
        wb = _blkdiag(w, z32)
        term = jnp.dot(acc_s[r] * scale[r], wb,
                       preferred_element_type=jnp.float32)
        contrib = term if contrib is None else contrib + term
    rootb = _blkdiag(root_ref[...], z32)
    bias = bias_ref[...]
    biaspk = jnp.concatenate([bias, bias, bias, bias], axis=1)
    h4 = jnp.tanh(
        jnp.dot(h3_ref[...], rootb, preferred_element_type=jnp.float32)
        + biaspk + contrib)

    hs = [h1_ref[...], h2_ref[...], h3_ref[...], h4]
    z1024 = jnp.zeros((32, 4 * H), jnp.float32)
    wih = wih0_ref[...]
    G = None
    for j in range(8):
        hp = hs[j % 4]
        zj = hp[0:128] if j < 4 else hp[128:256]
        wj = wih[32 * j:32 * (j + 1), :]
        wexp = _blkdiag(wj, z1024)
        term = jnp.dot(zj, wexp, preferred_element_type=jnp.float32)
        G = term if G is None else G + term
    b0 = b0_ref[...]
    for q in range(4):
        gx3_ref[:, q, :] = G[:, 1024 * q:1024 * (q + 1)] + b0

    def lstm_update(g, h, c):
        s = jax.nn.sigmoid(g[:, 0:3 * H])
        gg = jnp.tanh(g[:, 3 * H:4 * H])
        c = s[:, H:2 * H] * c + s[:, 0:H] * gg
        h = s[:, 2 * H:3 * H] * jnp.tanh(c)
        return h, c

    def step1(t, carry):
        h, c = carry
        t4 = lax.div(t, 4)
        tm = lax.rem(t, 4)
        g = gx3_ref[pl.ds(t4, 1), pl.ds(tm, 1), :].reshape(1, 4 * H)
        g = g + jnp.dot(h, whh0_ref[...], preferred_element_type=jnp.float32)
        h, c = lstm_update(g, h, c)
        ys_ref[pl.ds(t, 1), :] = h
        return (h, c)

    zh = jnp.zeros((1, H), jnp.float32)
    lax.fori_loop(0, B, step1, (zh, zh))

    gx2_ref[...] = (
        jnp.dot(ys_ref[...], wih1_ref[...], preferred_element_type=jnp.float32)
        + b1_ref[...])

    def step2(t, carry):
        h, c = carry
        g = gx2_ref[pl.ds(t, 1), :] + jnp.dot(
            h, whh1_ref[...], preferred_element_type=jnp.float32)
        h, c = lstm_update(g, h, c)
        ys_ref[pl.ds(t, 1), :] = h
        return (h, c)

    lax.fori_loop(0, B, step2, (zh, zh))

    y = jnp.maximum(
        jnp.dot(ys_ref[...], l1w_ref[...], preferred_element_type=jnp.float32)
        + l1b_ref[...], 0.0)
    out_ref[...] = (
        jnp.dot(y, l2w_ref[...], preferred_element_type=jnp.float32)
        + l2b_ref[...])


def _head(h1s, h2s, h3s, accs, scales, comp, bases, root, bias_row,
          wihT0, whhT0, b0_row, wihT1, whhT1, b1_row,
          lin1_wT, lin1_b_row, lin2_wT, lin2_b_row):
    return pl.pallas_call(
        _head_body,
        out_shape=jax.ShapeDtypeStruct((B, 1), jnp.float32),
        in_specs=[pl.BlockSpec(memory_space=pltpu.MemorySpace.SMEM)
                  if i == 5 else pl.BlockSpec()
                  for i in range(19)],
        scratch_shapes=[
            pltpu.VMEM((128, 4, 4 * H), jnp.float32),
            pltpu.VMEM((B, 4 * H), jnp.float32),
            pltpu.VMEM((B, H), jnp.float32),
        ],
    )(h1s, h2s, h3s, accs, scales, comp, bases, root, bias_row,
      wihT0, whhT0, b0_row, wihT1, whhT1, b1_row,
      lin1_wT, lin1_b_row, lin2_wT, lin2_b_row)


def kernel(x, edge_index, edge_type, batch,
           comp0, bases0, root0, bias0,
           comp1, bases1, root1, bias1,
           comp2, bases2, root2, bias2,
           comp3, bases3, root3, bias3,
           w_ih0, w_hh0, b_ih0, b_hh0,
           w_ih1, w_hh1, b_ih1, b_hh1,
           lin1_w, lin1_b, lin2_w, lin2_b):
    del batch

    src = edge_index[0].reshape(EROWS, 128)
    dst = edge_index[1].reshape(EROWS, 128)
    et = edge_type.reshape(EROWS, 128)
    padr = EROWS_PAD - EROWS
    src_p = jnp.pad(src, ((0, padr), (0, 0)))
    et_p = jnp.pad(et, ((0, padr), (0, 0)))
    dst_p = jnp.pad(dst, ((0, padr), (0, 0)), constant_values=N)
    zeros32 = jnp.zeros((TILE_ROWS, 32), jnp.float32)
    ones32 = jnp.ones((128, 32), jnp.float32)

    i0src, isrc, idst = _edge_indices(src_p, dst_p, et_p)

    cnt = _sc_count(idst, ones32, zeros32)
    cntp = cnt.reshape(2, R, PK, 128)

    hall0, self0 = _layer0(x, comp0, bases0, root0, bias0.reshape(1, 32))
    tab0 = hall0.reshape(T5, 32)
    self0p = self0.reshape(PK, 128)

    acc0 = _sc_edge(tab0, i0src, idst, zeros32)
    h1p, scalep = _post1(self0p, acc0.reshape(1, R, PK, 128), cntp)

    acc1 = _sc_edge(h1p.reshape(NPK, 32), isrc, idst, zeros32)
    h2p = _post23(h1p, acc1.reshape(1, R, PK, 128), scalep,
                  comp1, bases1, root1, bias1.reshape(1, 32))

    acc2 = _sc_edge(h2p.reshape(NPK, 32), isrc, idst, zeros32)
    h3p = _post23(h2p, acc2.reshape(1, R, PK, 128), scalep,
                  comp2, bases2, root2, bias2.reshape(1, 32))

    acc3 = _sc_edge(h3p.reshape(NPK, 32), isrc, idst, zeros32)
    accp3 = acc3.reshape(1, R, PK, 128)

    def perm_gates(w):
        return jnp.concatenate([w[:2 * H], w[3 * H:], w[2 * H:3 * H]], axis=0)

    PBH = 2 * B // 4
    out = _head(
        h1p[:PBH], h2p[:PBH], h3p[:PBH],
        accp3[:, :, :PBH], scalep[:, :PBH],
        comp3, bases3, root3, bias3.reshape(1, 32),
        perm_gates(w_ih0).T, perm_gates(w_hh0).T,
        perm_gates(b_ih0 + b_hh0).reshape(1, 4 * H),
        perm_gates(w_ih1).T, perm_gates(w_hh1).T,
        perm_gates(b_ih1 + b_hh1).reshape(1, 4 * H),
        lin1_w.T, lin1_b.reshape(1, 128), lin2_w.T, lin2_b.reshape(1, 1))
    return out[:, 0]


# speedup vs baseline: 1.2130x; 1.2130x over previous
import functools

import jax
import jax.numpy as jnp
from jax import lax
from jax.experimental import pallas as pl
from jax.experimental.pallas import tpu as pltpu
from jax.experimental.pallas import tpu_sc as plsc

N = 10000
E = 320000
D = 128
R = 5
B = 512
H = 256

NPK = 10240
PK = NPK // 4
T5 = R * NPK
EROWS = E // 128
EROWS_PAD = 2560
ROWS_PER_W = 80
TILE_ROWS = T5 // 16
WB_ROWS = NPK // 16
PB = 256
NBLK = PK // PB


@functools.cache
def _sc_mesh():
    return plsc.VectorSubcoreMesh(core_axis_name="c", subcore_axis_name="s")


def _idx_body(src_ref, dst_ref, et_ref, i0_ref, isrc_ref, idst_ref):
    et = et_ref[...]
    i0_ref[...] = et * NPK + src_ref[...]
    isrc_ref[...] = src_ref[...]
    idst_ref[...] = et * NPK + dst_ref[...]


def _edge_indices(src_p, dst_p, et_p):
    return pl.pallas_call(
        _idx_body,
        out_shape=(
            jax.ShapeDtypeStruct((EROWS_PAD, 128), jnp.int32),
            jax.ShapeDtypeStruct((EROWS_PAD, 128), jnp.int32),
            jax.ShapeDtypeStruct((EROWS_PAD, 128), jnp.int32),
        ),
    )(src_p, dst_p, et_p)


def _basis_w(comp_ref, bases_ref, r):
    w = comp_ref[r, 0] * bases_ref[0]
    for b in range(1, 4):
        w = w + comp_ref[r, b] * bases_ref[b]
    return w


def _layer0_body(x_ref, comp_ref, bases_ref, root_ref, bias_ref,
                 hall_ref, self_ref):
    x = x_ref[...]
    for r in range(R):
        w = _basis_w(comp_ref, bases_ref, r)
        hall_ref[r] = jnp.dot(x, w, preferred_element_type=jnp.float32)
    self_ref[...] = (
        jnp.dot(x, root_ref[...], preferred_element_type=jnp.float32)
        + bias_ref[...])


def _layer0(x, comp, bases, root, bias_row, rb=2048):
    return pl.pallas_call(
        _layer0_body,
        grid=(NPK // rb,),
        in_specs=[
            pl.BlockSpec((rb, D), lambda i: (i, 0)),
            pl.BlockSpec(memory_space=pltpu.MemorySpace.SMEM),
            pl.BlockSpec((4, D, 32), lambda i: (0, 0, 0)),
            pl.BlockSpec((D, 32), lambda i: (0, 0)),
            pl.BlockSpec((1, 32), lambda i: (0, 0)),
        ],
        out_specs=(
            pl.BlockSpec((R, rb, 32), lambda i: (0, i, 0)),
            pl.BlockSpec((rb, 32), lambda i: (i, 0)),
        ),
        out_shape=(
            jax.ShapeDtypeStruct((R, NPK, 32), jnp.float32),
            jax.ShapeDtypeStruct((NPK, 32), jnp.float32),
        ),
    )(x, comp, bases, root, bias_row)


def _sc_count(idst, ones32, zeros32):
    @functools.partial(
        pl.kernel,
        out_type=jax.ShapeDtypeStruct((2, R, NPK, 32), jnp.float32),
        mesh=_sc_mesh(),
        scratch_types=[
            pltpu.VMEM((ROWS_PER_W, 128), jnp.int32),
            pltpu.VMEM((128, 32), jnp.float32),
            pltpu.VMEM_SHARED((T5, 32), jnp.float32),
            pltpu.SemaphoreType.DMA,
            pltpu.SemaphoreType.DMA,
        ],
        compiler_params=pltpu.CompilerParams(use_tc_tiling_on_sc=False),
    )
    def k(idst_hbm, ones_hbm, zeros_hbm, out_hbm, idst_v, ones_v, cnt_sp,
          sem, ssem):
        cid = lax.axis_index("c")
        sid = lax.axis_index("s")
        wid = sid * 2 + cid
        pltpu.async_copy(zeros_hbm, cnt_sp.at[pl.ds(sid * TILE_ROWS, TILE_ROWS)],
                         sem).wait()
        pltpu.async_copy(idst_hbm.at[pl.ds(wid * ROWS_PER_W, ROWS_PER_W)],
                         idst_v, sem).wait()
        pltpu.async_copy(ones_hbm, ones_v, sem).wait()
        plsc.subcore_barrier()

        def body(j, carry):
            for u in range(5):
                pltpu.async_copy(ones_v, cnt_sp.at[idst_v.at[j * 5 + u]], ssem,
                                 add=True)
            return carry

        lax.fori_loop(0, ROWS_PER_W // 5, body, 0)

        def drain(j, carry):
            pltpu.make_async_copy(ones_hbm, ones_v, ssem).wait()
            return carry

        lax.fori_loop(0, ROWS_PER_W, drain, 0)
        plsc.subcore_barrier()
        for r in range(R):
            pltpu.async_copy(
                cnt_sp.at[pl.ds(r * NPK + sid * WB_ROWS, WB_ROWS)],
                out_hbm.at[cid, r].at[pl.ds(sid * WB_ROWS, WB_ROWS)],
                sem).wait()

    return k(idst, ones32, zeros32)


def _sc_edge(tab, idxg, idst, zeros32):
    @functools.partial(
        pl.kernel,
        out_type=jax.ShapeDtypeStruct((2, R, NPK, 32), jnp.float32),
        mesh=_sc_mesh(),
        scratch_types=[
            pltpu.VMEM((20, 128), jnp.int32),
            pltpu.VMEM((20, 128), jnp.int32),
            [pltpu.VMEM((128, 32), jnp.float32) for _ in range(5)],
            [pltpu.SemaphoreType.DMA for _ in range(5)],
            [pltpu.SemaphoreType.DMA for _ in range(5)],
            pltpu.VMEM_SHARED((T5, 32), jnp.float32),
            pltpu.SemaphoreType.DMA,
        ],
        compiler_params=pltpu.CompilerParams(use_tc_tiling_on_sc=False),
    )
    def k(tab_hbm, idxg_hbm, idst_hbm, zeros_hbm, out_hbm,
          idxg_v, idst_v, msgs, gsems, ssems, k_acc, sem):
        cid = lax.axis_index("c")
        sid = lax.axis_index("s")
        base = lax.select(cid == 0, sid * 120, 16 * 120 + sid * 40)
        trips = lax.select(cid == 0, 24, 8)

        def stage(k4):
            a = pltpu.async_copy(
                idxg_hbm.at[pl.ds(base + k4 * 20, 20)], idxg_v, sem)
            b = pltpu.async_copy(
                idst_hbm.at[pl.ds(base + k4 * 20, 20)], idst_v, sem)
            a.wait()
            b.wait()

        def swait(sem):
            pltpu.make_async_copy(zeros_hbm.at[pl.ds(0, 128)], msgs[0],
                                  sem).wait()

        def fire_gathers(rk):
            for j in range(5):
                pltpu.async_copy(tab_hbm.at[idxg_v.at[rk * 5 + j]], msgs[j],
                                 gsems[j])

        def fire_scatters(rk):
            for j in range(5):
                swait(gsems[j])
                pltpu.async_copy(msgs[j], k_acc.at[idst_v.at[rk * 5 + j]],
                                 ssems[j], add=True)

        zero_desc = pltpu.async_copy(
            zeros_hbm, k_acc.at[pl.ds(sid * TILE_ROWS, TILE_ROWS)], sem)
        stage(0)
        fire_gathers(0)
        zero_desc.wait()
        plsc.subcore_barrier()
        fire_scatters(0)

        def body(kk, carry):
            rk = lax.rem(kk, 4)

            @pl.when(rk == 0)
            def _():
                for j in range(5):
                    swait(ssems[j])
                stage(lax.div(kk, 4))

            for j in range(5):
                @pl.when(rk != 0)
                def _(j=j):
                    swait(ssems[j])
            fire_gathers(rk)
            fire_scatters(rk)
            return carry

        lax.fori_loop(1, trips, body, 0)
        for j in range(5):
            swait(ssems[j])
        plsc.subcore_barrier()
        wdescs = [
            pltpu.async_copy(
                k_acc.at[pl.ds(r * NPK + sid * WB_ROWS, WB_ROWS)],
                out_hbm.at[cid, r].at[pl.ds(sid * WB_ROWS, WB_ROWS)],
                sem)
            for r in range(R)
        ]
        for d in wdescs:
            d.wait()

    return k(tab, idxg, idst, zeros32)


def _blkdiag(w, zeros):
    rows = [jnp.concatenate([w if q == q2 else zeros for q in range(4)], axis=1)
            for q2 in range(4)]
    return jnp.concatenate(rows, axis=0)


def _post1_body(self_ref, acc_ref, cnt_ref, h_ref, scale_ref):
    cn = cnt_ref[...]
    scale = 1.0 / jnp.maximum(cn[0] + cn[1], 1.0)
    scale_ref[...] = scale
    a = acc_ref[...]
    acc_s = a[0] + a[1]
    agg = (acc_s * scale).sum(axis=0)
    h_ref[...] = jnp.tanh(self_ref[...] + agg)


def _post1(self0p, accp, cntp):
    return pl.pallas_call(
        _post1_body,
        grid=(NBLK,),
        in_specs=[
            pl.BlockSpec((PB, 128), lambda i: (i, 0)),
            pl.BlockSpec((2, R, PB, 128), lambda i: (0, 0, i, 0)),
            pl.BlockSpec((2, R, PB, 128), lambda i: (0, 0, i, 0)),
        ],
        out_specs=(
            pl.BlockSpec((PB, 128), lambda i: (i, 0)),
            pl.BlockSpec((R, PB, 128), lambda i: (0, i, 0)),
        ),
        out_shape=(
            jax.ShapeDtypeStruct((PK, 128), jnp.float32),
            jax.ShapeDtypeStruct((R, PK, 128), jnp.float32),
        ),
    )(self0p, accp, cntp)


def _post23_body(hprev_ref, acc_ref, scale_ref, comp_ref, bases_ref,
                 root_ref, bias_ref, h_ref):
    z32 = jnp.zeros((32, 32), jnp.float32)
    a = acc_ref[...]
    acc_s = a[0] + a[1]
    scale = scale_ref[...]
    contrib = None
    for r in range(R):
        w = _basis_w(comp_ref, bases_ref, r)
        wb = _blkdiag(w, z32)
        term = jnp.dot(acc_s[r] * scale[r], wb,
                       preferred_element_type=jnp.float32)
        contrib = term if contrib is None else contrib + term
    rootb = _blkdiag(root_ref[...], z32)
    bias = bias_ref[...]
    biaspk = jnp.concatenate([bias, bias, bias, bias], axis=1)
    h_ref[...] = jnp.tanh(
        jnp.dot(hprev_ref[...], rootb, preferred_element_type=jnp.float32)
        + biaspk + contrib)


def _post23(hprevp, accp, scalep, comp, bases, root, bias_row):
    return pl.pallas_call(
        _post23_body,
        grid=(NBLK,),
        in_specs=[
            pl.BlockSpec((PB, 128), lambda i: (i, 0)),
            pl.BlockSpec((2, R, PB, 128), lambda i: (0, 0, i, 0)),
            pl.BlockSpec((R, PB, 128), lambda i: (0, i, 0)),
            pl.BlockSpec(memory_space=pltpu.MemorySpace.SMEM),
            pl.BlockSpec((4, 32, 32), lambda i: (0, 0, 0)),
            pl.BlockSpec((32, 32), lambda i: (0, 0)),
            pl.BlockSpec((1, 32), lambda i: (0, 0)),
        ],
        out_specs=pl.BlockSpec((PB, 128), lambda i: (i, 0)),
        out_shape=jax.ShapeDtypeStruct((PK, 128), jnp.float32),
    )(hprevp, accp, scalep, comp, bases, root, bias_row)


def _head_body(h1_ref, h2_ref, h3_ref, acc_ref, scale_ref,
               comp_ref, bases_ref, root_ref, bias_ref,
               wih0_ref, whh0_ref, b0_ref,
               wih1_ref, whh1_ref, b1_ref,
               l1w_ref, l1b_ref, l2w_ref, l2b_ref,
               out_ref, gx3_ref, gx2_ref, ys_ref):
    z32 = jnp.zeros((32, 32), jnp.float32)
    a = acc_ref[...]
    acc_s = a[0] + a[1]
    scale = scale_ref[...]
    contrib = None
    for r in range(R):
        w = _basis_w(comp_ref, bases_ref, r)
        wb = _blkdiag(w, z32)
        term = jnp.dot(acc_s[r] * scale[r], wb,
                       preferred_element_type=jnp.float32)
        contrib = term if contrib is None else contrib + term
    rootb = _blkdiag(root_ref[...], z32)
    bias = bias_ref[...]
    biaspk = jnp.concatenate([bias, bias, bias, bias], axis=1)
    h4 = jnp.tanh(
        jnp.dot(h3_ref[...], rootb, preferred_element_type=jnp.float32)
        + biaspk + contrib)

    hs = [h1_ref[...], h2_ref[...], h3_ref[...], h4]
    z1024 = jnp.zeros((32, 4 * H), jnp.float32)
    wih = wih0_ref[...]
    G = None
    for j in range(8):
        hp = hs[j % 4]
        zj = hp[0:128] if j < 4 else hp[128:256]
        wj = wih[32 * j:32 * (j + 1), :]
        wexp = _blkdiag(wj, z1024)
        term = jnp.dot(zj, wexp, preferred_element_type=jnp.float32)
        G = term if G is None else G + term
    b0 = b0_ref[...]
    for q in range(4):
        gx3_ref[:, q, :] = G[:, 1024 * q:1024 * (q + 1)] + b0

    def lstm_update(g, h, c):
        s = jax.nn.sigmoid(g[:, 0:3 * H])
        gg = jnp.tanh(g[:, 3 * H:4 * H])
        c = s[:, H:2 * H] * c + s[:, 0:H] * gg
        h = s[:, 2 * H:3 * H] * jnp.tanh(c)
        return h, c

    def step1(t, carry):
        h, c = carry
        t4 = lax.div(t, 4)
        tm = lax.rem(t, 4)
        g = gx3_ref[pl.ds(t4, 1), pl.ds(tm, 1), :].reshape(1, 4 * H)
        g = g + jnp.dot(h, whh0_ref[...], preferred_element_type=jnp.float32)
        h, c = lstm_update(g, h, c)
        ys_ref[pl.ds(t, 1), :] = h
        return (h, c)

    zh = jnp.zeros((1, H), jnp.float32)
    lax.fori_loop(0, B, step1, (zh, zh))

    gx2_ref[...] = (
        jnp.dot(ys_ref[...], wih1_ref[...], preferred_element_type=jnp.float32)
        + b1_ref[...])

    def step2(t, carry):
        h, c = carry
        g = gx2_ref[pl.ds(t, 1), :] + jnp.dot(
            h, whh1_ref[...], preferred_element_type=jnp.float32)
        h, c = lstm_update(g, h, c)
        ys_ref[pl.ds(t, 1), :] = h
        return (h, c)

    lax.fori_loop(0, B, step2, (zh, zh))

    y = jnp.maximum(
        jnp.dot(ys_ref[...], l1w_ref[...], preferred_element_type=jnp.float32)
        + l1b_ref[...], 0.0)
    out_ref[...] = (
        jnp.dot(y, l2w_ref[...], preferred_element_type=jnp.float32)
        + l2b_ref[...])


def _head(h1s, h2s, h3s, accs, scales, comp, bases, root, bias_row,
          wihT0, whhT0, b0_row, wihT1, whhT1, b1_row,
          lin1_wT, lin1_b_row, lin2_wT, lin2_b_row):
    return pl.pallas_call(
        _head_body,
        out_shape=jax.ShapeDtypeStruct((B, 1), jnp.float32),
        in_specs=[pl.BlockSpec(memory_space=pltpu.MemorySpace.SMEM)
                  if i == 5 else pl.BlockSpec()
                  for i in range(19)],
        scratch_shapes=[
            pltpu.VMEM((128, 4, 4 * H), jnp.float32),
            pltpu.VMEM((B, 4 * H), jnp.float32),
            pltpu.VMEM((B, H), jnp.float32),
        ],
    )(h1s, h2s, h3s, accs, scales, comp, bases, root, bias_row,
      wihT0, whhT0, b0_row, wihT1, whhT1, b1_row,
      lin1_wT, lin1_b_row, lin2_wT, lin2_b_row)


def kernel(x, edge_index, edge_type, batch,
           comp0, bases0, root0, bias0,
           comp1, bases1, root1, bias1,
           comp2, bases2, root2, bias2,
           comp3, bases3, root3, bias3,
           w_ih0, w_hh0, b_ih0, b_hh0,
           w_ih1, w_hh1, b_ih1, b_hh1,
           lin1_w, lin1_b, lin2_w, lin2_b):
    del batch

    src = edge_index[0].reshape(EROWS, 128)
    dst = edge_index[1].reshape(EROWS, 128)
    et = edge_type.reshape(EROWS, 128)
    padr = EROWS_PAD - EROWS
    src_p = jnp.pad(src, ((0, padr), (0, 0)))
    et_p = jnp.pad(et, ((0, padr), (0, 0)))
    dst_p = jnp.pad(dst, ((0, padr), (0, 0)), constant_values=N)
    zeros32 = jnp.zeros((TILE_ROWS, 32), jnp.float32)
    ones32 = jnp.ones((128, 32), jnp.float32)

    i0src, isrc, idst = _edge_indices(src_p, dst_p, et_p)

    cnt = _sc_count(idst, ones32, zeros32)
    cntp = cnt.reshape(2, R, PK, 128)

    hall0, self0 = _layer0(x, comp0, bases0, root0, bias0.reshape(1, 32))
    tab0 = hall0.reshape(T5, 32)
    self0p = self0.reshape(PK, 128)

    acc0 = _sc_edge(tab0, i0src, idst, zeros32)
    h1p, scalep = _post1(self0p, acc0.reshape(2, R, PK, 128), cntp)

    acc1 = _sc_edge(h1p.reshape(NPK, 32), isrc, idst, zeros32)
    h2p = _post23(h1p, acc1.reshape(2, R, PK, 128), scalep,
                  comp1, bases1, root1, bias1.reshape(1, 32))

    acc2 = _sc_edge(h2p.reshape(NPK, 32), isrc, idst, zeros32)
    h3p = _post23(h2p, acc2.reshape(2, R, PK, 128), scalep,
                  comp2, bases2, root2, bias2.reshape(1, 32))

    acc3 = _sc_edge(h3p.reshape(NPK, 32), isrc, idst, zeros32)
    accp3 = acc3.reshape(2, R, PK, 128)

    def perm_gates(w):
        return jnp.concatenate([w[:2 * H], w[3 * H:], w[2 * H:3 * H]], axis=0)

    PBH = 2 * B // 4
    out = _head(
        h1p[:PBH], h2p[:PBH], h3p[:PBH],
        accp3[:, :, :PBH], scalep[:, :PBH],
        comp3, bases3, root3, bias3.reshape(1, 32),
        perm_gates(w_ih0).T, perm_gates(w_hh0).T,
        perm_gates(b_ih0 + b_hh0).reshape(1, 4 * H),
        perm_gates(w_ih1).T, perm_gates(w_hh1).T,
        perm_gates(b_ih1 + b_hh1).reshape(1, 4 * H),
        lin1_w.T, lin1_b.reshape(1, 128), lin2_w.T, lin2_b.reshape(1, 1))
    return out[:, 0]
